# baseline (speedup 1.0000x reference)
import jax
import jax.numpy as jnp
from jax import lax
from jax.experimental import pallas as pl
from jax.experimental.pallas import tpu as pltpu
from jax.experimental.pallas import tpu_sc as plsc

N_NODE = 10000
D_IN = 128
E_RAW = 320000
BATCH = 4096

NC = 2
NS = 16
NW = NC * NS
CHUNK = 128

K2 = 160
NCHUNK = NS * K2
E_PAD = NCHUNK * CHUNK
IBLK = 16
NBLK = K2 // IBLK
ACC_ROWS = 10112
RPT = ACC_ROWS // NS
DUMMY = N_NODE


def _mesh():
    return plsc.VectorSubcoreMesh(
        core_axis_name="c", subcore_axis_name="s", num_cores=NC, num_subcores=NS)


def _pipelined_gather_scatter(tbl, acc, src3, dst3, idx_s, idx_d,
                              buf0, buf1, sem0, sem1, ss0, ss1,
                              st0, st1, s):
    def issue(j_blk, cur, buf, sem):
        pltpu.async_copy(tbl.at[idx_s.at[cur, j_blk, 0]], buf, sem)

    def wait(j_blk, cur, buf, sem):
        pltpu.make_async_copy(tbl.at[idx_s.at[cur, j_blk, 0]], buf, sem).wait()

    def s_issue(j_blk, cur, buf, sem):
        pltpu.async_copy(buf, acc.at[idx_d.at[cur, j_blk, 0]], sem, add=True)

    def s_wait(j_blk, cur, buf, sem):
        pltpu.make_async_copy(
            buf, acc.at[idx_d.at[cur, j_blk, 0]], sem).wait()

    pltpu.sync_copy(src3.at[pl.ds(s * K2, IBLK)], idx_s.at[0])
    pltpu.sync_copy(dst3.at[pl.ds(s * K2, IBLK)], idx_d.at[0])
    issue(0, 0, buf0, sem0)

    for blk in range(NBLK):
        cur = blk % 2
        nxt = 1 - cur
        if blk + 1 < NBLK:
            off = s * K2 + (blk + 1) * IBLK
            pltpu.async_copy(src3.at[pl.ds(off, IBLK)], idx_s.at[nxt], st0)
            pltpu.async_copy(dst3.at[pl.ds(off, IBLK)], idx_d.at[nxt], st1)

        def pair(t, carry):
            wait(2 * t, cur, buf0, sem0)
            s_issue(2 * t, cur, buf0, ss0)

            if blk > 0:
                s_wait(2 * t, cur, buf1, ss1)
            else:
                @pl.when(t > 0)
                def _():
                    s_wait(2 * t, cur, buf1, ss1)
            issue(2 * t + 1, cur, buf1, sem1)
            wait(2 * t + 1, cur, buf1, sem1)
            s_issue(2 * t + 1, cur, buf1, ss1)
            s_wait(2 * t, cur, buf0, ss0)

            @pl.when(t < IBLK // 2 - 1)
            def _():
                issue(2 * t + 2, cur, buf0, sem0)
            return carry
        lax.fori_loop(0, IBLK // 2, pair, 0)

        if blk + 1 < NBLK:
            off = s * K2 + (blk + 1) * IBLK
            pltpu.make_async_copy(
                src3.at[pl.ds(off, IBLK)], idx_s.at[nxt], st0).wait()
            pltpu.make_async_copy(
                dst3.at[pl.ds(off, IBLK)], idx_d.at[nxt], st1).wait()
            issue(0, nxt, buf0, sem0)
    s_wait(IBLK - 1, (NBLK - 1) % 2, buf1, ss1)


def _make_sc_pass_a():
    return pl.kernel(
        _sc_pass_a_body,
        out_type=(
            jax.ShapeDtypeStruct((ACC_ROWS, D_IN), jnp.float32),
            jax.ShapeDtypeStruct((ACC_ROWS, 128), jnp.float32),
            jax.ShapeDtypeStruct((BATCH, D_IN), jnp.float32),
        ),
        mesh=_mesh(),
        scratch_types=[
            pltpu.VMEM_SHARED((ACC_ROWS, D_IN), jnp.float32),
            pltpu.VMEM((2, IBLK, 1, CHUNK), jnp.int32),
            pltpu.VMEM((2, IBLK, 1, CHUNK), jnp.int32),
            pltpu.VMEM((CHUNK, D_IN), jnp.float32),
            pltpu.VMEM((CHUNK, D_IN), jnp.float32),
            pltpu.VMEM((1, CHUNK), jnp.int32),
            pltpu.SemaphoreType.DMA,
            pltpu.SemaphoreType.DMA,
            pltpu.SemaphoreType.DMA,
            pltpu.SemaphoreType.DMA,
            pltpu.SemaphoreType.DMA,
            pltpu.SemaphoreType.DMA,
        ],
    )


def _sc_pass_a_body(cellx, src3, dst3, drugx, drug3, ones128, zrow,
                    seg_out, cnt_out, dx_out,
                    acc_sh, idx_s, idx_d, buf0, buf1, didx,
                    sem0, sem1, ss0, ss1, st0, st1):
    c = lax.axis_index("c")
    s = lax.axis_index("s")

    base = s * RPT
    pltpu.sync_copy(zrow, acc_sh.at[pl.ds(base, RPT)])
    plsc.subcore_barrier()

    @pl.when(c == 0)
    def _():
        _pipelined_gather_scatter(cellx, acc_sh, src3, dst3, idx_s, idx_d,
                                  buf0, buf1, sem0, sem1, ss0, ss1,
                                  st0, st1, s)

    @pl.when(c == 1)
    def _():
        pltpu.sync_copy(ones128, buf0)
        for blk in range(NBLK):
            pltpu.sync_copy(dst3.at[pl.ds(s * K2 + blk * IBLK, IBLK)],
                            idx_d.at[0])
            for u in range(IBLK):
                pltpu.async_copy(buf0, acc_sh.at[idx_d.at[0, u, 0]], sem0,
                                 add=True)
            for u in range(IBLK):
                pltpu.make_async_copy(
                    buf0, acc_sh.at[idx_d.at[0, u, 0]], sem0).wait()
        for t in range(2):
            ch = s * 2 + t
            pltpu.sync_copy(drug3.at[ch], didx)
            pltpu.async_copy(drugx.at[didx.at[0]], buf1, sem1).wait()
            pltpu.sync_copy(buf1, dx_out.at[pl.ds(ch * CHUNK, CHUNK)])

    plsc.subcore_barrier()

    @pl.when(c == 0)
    def _():
        pltpu.sync_copy(acc_sh.at[pl.ds(base, RPT)],
                        seg_out.at[pl.ds(base, RPT)])

    @pl.when(c == 1)
    def _():
        pltpu.sync_copy(acc_sh.at[pl.ds(base, RPT)],
                        cnt_out.at[pl.ds(base, RPT)])


def _make_sc_pass_c():
    return pl.kernel(
        _sc_pass_c_body,
        out_type=(
            jax.ShapeDtypeStruct((NC, BATCH, 128), jnp.float32),
            jax.ShapeDtypeStruct((BATCH, 256), jnp.float32),
            jax.ShapeDtypeStruct((BATCH, 128), jnp.float32),
        ),
        mesh=_mesh(),
        scratch_types=[
            pltpu.VMEM_SHARED((ACC_ROWS, 128), jnp.float32),
            pltpu.VMEM((2, IBLK, 1, CHUNK), jnp.int32),
            pltpu.VMEM((2, IBLK, 1, CHUNK), jnp.int32),
            pltpu.VMEM((CHUNK, 128), jnp.float32),
            pltpu.VMEM((CHUNK, 128), jnp.float32),
            pltpu.VMEM((16, 256), jnp.float32),
            pltpu.VMEM((1, CHUNK), jnp.int32),
            pltpu.VMEM((1, 16), jnp.int32),
            pltpu.SemaphoreType.DMA,
            pltpu.SemaphoreType.DMA,
            pltpu.SemaphoreType.DMA,
            pltpu.SemaphoreType.DMA,
            pltpu.SemaphoreType.DMA,
            pltpu.SemaphoreType.DMA,
        ],
    )


def _sc_pass_c_body(p2a, p2b, src3, dst3, q, cnt128, cell3, cell16, zrow,
                    aggb_out, qb_out, cntb_out,
                    acc_sh, idx_s, idx_d, buf0, buf1, qbuf, cidx, cidx16,
                    sem0, sem1, ss0, ss1, st0, st1):
    c = lax.axis_index("c")
    s = lax.axis_index("s")
    w = c * NS + s

    base = s * RPT
    pltpu.sync_copy(zrow, acc_sh.at[pl.ds(base, RPT)])
    plsc.subcore_barrier()

    @pl.when(c == 0)
    def _():
        _pipelined_gather_scatter(p2a, acc_sh, src3, dst3, idx_s, idx_d,
                                  buf0, buf1, sem0, sem1, ss0, ss1,
                                  st0, st1, s)

    @pl.when(c == 1)
    def _():
        _pipelined_gather_scatter(p2b, acc_sh, src3, dst3, idx_s, idx_d,
                                  buf0, buf1, sem0, sem1, ss0, ss1,
                                  st0, st1, s)

    def qc_body(t, carry):
        pltpu.sync_copy(cell16.at[w * 8 + t], cidx16)
        pltpu.async_copy(q.at[cidx16.at[0]], qbuf, sem0).wait()
        pltpu.sync_copy(qbuf, qb_out.at[pl.ds(w * CHUNK + t * 16, 16)])
        pltpu.async_copy(cnt128.at[cidx16.at[0]], buf0.at[pl.ds(0, 16)],
                         sem0).wait()
        pltpu.sync_copy(buf0.at[pl.ds(0, 16)],
                        cntb_out.at[pl.ds(w * CHUNK + t * 16, 16)])
        return carry
    lax.fori_loop(0, 8, qc_body, 0)

    plsc.subcore_barrier()

    for t in range(2):
        ch = s * 2 + t
        pltpu.sync_copy(cell3.at[ch], cidx)
        pltpu.async_copy(acc_sh.at[cidx.at[0]], buf1, sem1).wait()
        pltpu.sync_copy(buf1, aggb_out.at[c, pl.ds(ch * CHUNK, CHUNK)])


def _tc_b_body(seg_ref, cnt_ref, cx_ref, wl1_ref, wr1_ref, b1_ref,
               wl2a_ref, wl2b_ref, wr2_ref, b2_ref,
               p2a_ref, p2b_ref, q_ref):
    cnt = jnp.maximum(cnt_ref[:, 0:1], 1.0)
    agg = seg_ref[...] / cnt
    h1 = jnp.dot(agg, wl1_ref[...], preferred_element_type=jnp.float32)
    h1 += jnp.dot(cx_ref[...], wr1_ref[...], preferred_element_type=jnp.float32)
    h1 = jnp.maximum(h1 + b1_ref[...], 0.0)
    p2a_ref[...] = jnp.dot(h1, wl2a_ref[...], preferred_element_type=jnp.float32)
    p2b_ref[...] = jnp.dot(h1, wl2b_ref[...], preferred_element_type=jnp.float32)
    q_ref[...] = (jnp.dot(h1, wr2_ref[...], preferred_element_type=jnp.float32)
                  + b2_ref[...])


def _tc_pass_b(seg, cnt128, cx_pad, wl1, wr1, b1, wl2a, wl2b, wr2, b2):
    R = 632
    n = ACC_ROWS // R
    return pl.pallas_call(
        _tc_b_body,
        grid=(n,),
        in_specs=[
            pl.BlockSpec((R, D_IN), lambda i: (i, 0)),
            pl.BlockSpec((R, 128), lambda i: (i, 0)),
            pl.BlockSpec((R, D_IN), lambda i: (i, 0)),
            pl.BlockSpec((D_IN, 1024), lambda i: (0, 0)),
            pl.BlockSpec((D_IN, 1024), lambda i: (0, 0)),
            pl.BlockSpec((1, 1024), lambda i: (0, 0)),
            pl.BlockSpec((1024, 128), lambda i: (0, 0)),
            pl.BlockSpec((1024, 128), lambda i: (0, 0)),
            pl.BlockSpec((1024, 256), lambda i: (0, 0)),
            pl.BlockSpec((1, 256), lambda i: (0, 0)),
        ],
        out_specs=[
            pl.BlockSpec((R, 128), lambda i: (i, 0)),
            pl.BlockSpec((R, 128), lambda i: (i, 0)),
            pl.BlockSpec((R, 256), lambda i: (i, 0)),
        ],
        out_shape=[
            jax.ShapeDtypeStruct((ACC_ROWS, 128), jnp.float32),
            jax.ShapeDtypeStruct((ACC_ROWS, 128), jnp.float32),
            jax.ShapeDtypeStruct((ACC_ROWS, 256), jnp.float32),
        ],
    )(seg, cnt128, cx_pad, wl1, wr1, b1, wl2a, wl2b, wr2, b2)


def _elu(x):
    return jnp.where(x > 0.0, x, jnp.exp(x) - 1.0)


def _tc_d_body(dx_ref, qb_ref, aggb_ref, cntb_ref,
               wde_ref, bde_ref,
               wg1_ref, bg1_ref, wg2_ref, bg2_ref, wg3_ref, bg3_ref,
               out_ref):
    dh = jnp.dot(dx_ref[...], wde_ref[...], preferred_element_type=jnp.float32)
    dh = jnp.maximum(dh + bde_ref[...], 0.0)
    cnt = jnp.maximum(cntb_ref[:, 0:1], 1.0)
    agg2 = jnp.concatenate([aggb_ref[0], aggb_ref[1]], axis=1) / cnt
    h2 = jnp.maximum(agg2 + qb_ref[...], 0.0)
    x = jnp.concatenate([dh, h2], axis=1)
    x = _elu(jnp.dot(x, wg1_ref[...], preferred_element_type=jnp.float32)
             + bg1_ref[...])
    x = _elu(jnp.dot(x, wg2_ref[...], preferred_element_type=jnp.float32)
             + bg2_ref[...])
    out_ref[...] = (jnp.dot(x, wg3_ref[...], preferred_element_type=jnp.float32)
                    + bg3_ref[...])


def _tc_pass_d(dx, qb, aggb, cntb, wde, bde,
               wg1, bg1, wg2, bg2, wg3p, bg3p):
    R = 512
    n = BATCH // R
    return pl.pallas_call(
        _tc_d_body,
        grid=(n,),
        in_specs=[
            pl.BlockSpec((R, D_IN), lambda i: (i, 0)),
            pl.BlockSpec((R, 256), lambda i: (i, 0)),
            pl.BlockSpec((NC, R, 128), lambda i: (0, i, 0)),
            pl.BlockSpec((R, 128), lambda i: (i, 0)),
            pl.BlockSpec((D_IN, 256), lambda i: (0, 0)),
            pl.BlockSpec((1, 256), lambda i: (0, 0)),
            pl.BlockSpec((512, 512), lambda i: (0, 0)),
            pl.BlockSpec((1, 512), lambda i: (0, 0)),
            pl.BlockSpec((512, 512), lambda i: (0, 0)),
            pl.BlockSpec((1, 512), lambda i: (0, 0)),
            pl.BlockSpec((512, 128), lambda i: (0, 0)),
            pl.BlockSpec((1, 128), lambda i: (0, 0)),
        ],
        out_specs=pl.BlockSpec((R, 128), lambda i: (i, 0)),
        out_shape=jax.ShapeDtypeStruct((BATCH, 128), jnp.float32),
    )(dx, qb, aggb, cntb, wde, bde, wg1, bg1, wg2, bg2, wg3p, bg3p)


@jax.jit
def kernel(drug_x, cell_x, W_demb, b_demb, Wl1, Wr1, b1, Wl2, Wr2, b2,
           Wg1, bg1, Wg2, bg2, Wg3, bg3, cell_edges, cell, drug):
    f32 = jnp.float32
    pad_e = E_PAD - E_RAW
    src3 = jnp.concatenate(
        [cell_edges[0], jnp.zeros((pad_e,), jnp.int32)]).reshape(NCHUNK, 1, CHUNK)
    dst3 = jnp.concatenate(
        [cell_edges[1], jnp.full((pad_e,), DUMMY, jnp.int32)]).reshape(NCHUNK, 1, CHUNK)
    drug3 = drug.reshape(NW, 1, CHUNK)
    cell3 = cell.reshape(NW, 1, CHUNK)
    cell16 = cell.reshape(NW * 8, 1, 16)
    cx_pad = jnp.concatenate(
        [cell_x, jnp.zeros((ACC_ROWS - N_NODE, D_IN), f32)], axis=0)
    zrow = jnp.zeros((RPT, D_IN), f32)
    ones128 = jnp.ones((CHUNK, 128), f32)
    wl2a = Wl2[:, :128]
    wl2b = Wl2[:, 128:]
    wg3p = jnp.concatenate([Wg3, jnp.zeros((512, 127), f32)], axis=1)
    bg3p = jnp.concatenate([bg3, jnp.zeros((127,), f32)]).reshape(1, 128)

    seg, cnt128, dx = _make_sc_pass_a()(cx_pad, src3, dst3, drug_x, drug3,
                                        ones128, zrow)

    p2a, p2b, q = _tc_pass_b(
        seg, cnt128, cx_pad, Wl1, Wr1, b1.reshape(1, 1024), wl2a, wl2b,
        Wr2, b2.reshape(1, 256))

    aggb, qb, cntb = _make_sc_pass_c()(
        p2a, p2b, src3, dst3, q, cnt128, cell3, cell16, zrow)

    out = _tc_pass_d(dx, qb, aggb, cntb,
                     W_demb, b_demb.reshape(1, 256),
                     Wg1, bg1.reshape(1, 512), Wg2, bg2.reshape(1, 512),
                     wg3p, bg3p)
    return out[:, :1]


# speedup vs baseline: 1.0468x; 1.0468x over previous
import jax
import jax.numpy as jnp
from jax import lax
from jax.experimental import pallas as pl
from jax.experimental.pallas import tpu as pltpu
from jax.experimental.pallas import tpu_sc as plsc

N_NODE = 10000
D_IN = 128
E_RAW = 320000
BATCH = 4096

NC = 2
NS = 16
NW = NC * NS
CHUNK = 128

K2 = 160
NCHUNK = NS * K2
E_PAD = NCHUNK * CHUNK
IBLK = 16
NBLK = K2 // IBLK
ACC_ROWS = 10112
RPT = ACC_ROWS // NS
DUMMY = N_NODE


def _mesh():
    return plsc.VectorSubcoreMesh(
        core_axis_name="c", subcore_axis_name="s", num_cores=NC, num_subcores=NS)


def _pipelined_gather_scatter(tbl, acc, src3, dst3, idx_s, idx_d,
                              buf0, buf1, sem0, sem1, ss0, ss1,
                              st0, st1, s):
    H = CHUNK // 2

    def issue(j_blk, cur, buf, sa, sb):
        pltpu.async_copy(tbl.at[idx_s.at[cur, j_blk, 0, pl.ds(0, H)]],
                         buf.at[pl.ds(0, H)], sa)
        pltpu.async_copy(tbl.at[idx_s.at[cur, j_blk, 0, pl.ds(H, H)]],
                         buf.at[pl.ds(H, H)], sb)

    def wait(j_blk, cur, buf, sa, sb):
        pltpu.make_async_copy(tbl.at[idx_s.at[cur, j_blk, 0, pl.ds(0, H)]],
                              buf.at[pl.ds(0, H)], sa).wait()
        pltpu.make_async_copy(tbl.at[idx_s.at[cur, j_blk, 0, pl.ds(H, H)]],
                              buf.at[pl.ds(H, H)], sb).wait()

    def scat(j_blk, cur, buf):
        pltpu.sync_copy(buf, acc.at[idx_d.at[cur, j_blk, 0]], add=True)

    pltpu.sync_copy(src3.at[pl.ds(s * K2, IBLK)], idx_s.at[0])
    pltpu.sync_copy(dst3.at[pl.ds(s * K2, IBLK)], idx_d.at[0])
    issue(0, 0, buf0, sem0, ss0)

    for blk in range(NBLK):
        cur = blk % 2
        nxt = 1 - cur
        if blk + 1 < NBLK:
            off = s * K2 + (blk + 1) * IBLK
            pltpu.async_copy(src3.at[pl.ds(off, IBLK)], idx_s.at[nxt], st0)
            pltpu.async_copy(dst3.at[pl.ds(off, IBLK)], idx_d.at[nxt], st1)

        def pair(t, carry):
            issue(2 * t + 1, cur, buf1, sem1, ss1)
            wait(2 * t, cur, buf0, sem0, ss0)
            scat(2 * t, cur, buf0)

            @pl.when(t < IBLK // 2 - 1)
            def _():
                issue(2 * t + 2, cur, buf0, sem0, ss0)
            wait(2 * t + 1, cur, buf1, sem1, ss1)
            scat(2 * t + 1, cur, buf1)
            return carry
        lax.fori_loop(0, IBLK // 2, pair, 0)

        if blk + 1 < NBLK:
            off = s * K2 + (blk + 1) * IBLK
            pltpu.make_async_copy(
                src3.at[pl.ds(off, IBLK)], idx_s.at[nxt], st0).wait()
            pltpu.make_async_copy(
                dst3.at[pl.ds(off, IBLK)], idx_d.at[nxt], st1).wait()
            issue(0, nxt, buf0, sem0, ss0)


def _make_sc_pass_a():
    return pl.kernel(
        _sc_pass_a_body,
        out_type=(
            jax.ShapeDtypeStruct((ACC_ROWS, D_IN), jnp.float32),
            jax.ShapeDtypeStruct((ACC_ROWS, 128), jnp.float32),
            jax.ShapeDtypeStruct((BATCH, D_IN), jnp.float32),
        ),
        mesh=_mesh(),
        scratch_types=[
            pltpu.VMEM_SHARED((ACC_ROWS, D_IN), jnp.float32),
            pltpu.VMEM((2, IBLK, 1, CHUNK), jnp.int32),
            pltpu.VMEM((2, IBLK, 1, CHUNK), jnp.int32),
            pltpu.VMEM((CHUNK, D_IN), jnp.float32),
            pltpu.VMEM((CHUNK, D_IN), jnp.float32),
            pltpu.VMEM((1, CHUNK), jnp.int32),
            pltpu.SemaphoreType.DMA,
            pltpu.SemaphoreType.DMA,
            pltpu.SemaphoreType.DMA,
            pltpu.SemaphoreType.DMA,
            pltpu.SemaphoreType.DMA,
            pltpu.SemaphoreType.DMA,
        ],
    )


def _sc_pass_a_body(cellx, src3, dst3, drugx, drug3, ones128, zrow,
                    seg_out, cnt_out, dx_out,
                    acc_sh, idx_s, idx_d, buf0, buf1, didx,
                    sem0, sem1, ss0, ss1, st0, st1):
    c = lax.axis_index("c")
    s = lax.axis_index("s")

    base = s * RPT
    pltpu.sync_copy(zrow, acc_sh.at[pl.ds(base, RPT)])
    plsc.subcore_barrier()

    @pl.when(c == 0)
    def _():
        _pipelined_gather_scatter(cellx, acc_sh, src3, dst3, idx_s, idx_d,
                                  buf0, buf1, sem0, sem1, ss0, ss1,
                                  st0, st1, s)

    @pl.when(c == 1)
    def _():
        pltpu.sync_copy(ones128, buf0)
        for blk in range(NBLK):
            pltpu.sync_copy(dst3.at[pl.ds(s * K2 + blk * IBLK, IBLK)],
                            idx_d.at[0])
            for u in range(IBLK):
                pltpu.async_copy(buf0, acc_sh.at[idx_d.at[0, u, 0]], sem0,
                                 add=True)
            for u in range(IBLK):
                pltpu.make_async_copy(
                    buf0, acc_sh.at[idx_d.at[0, u, 0]], sem0).wait()
        for t in range(2):
            ch = s * 2 + t
            pltpu.sync_copy(drug3.at[ch], didx)
            pltpu.async_copy(drugx.at[didx.at[0]], buf1, sem1).wait()
            pltpu.sync_copy(buf1, dx_out.at[pl.ds(ch * CHUNK, CHUNK)])

    plsc.subcore_barrier()

    @pl.when(c == 0)
    def _():
        pltpu.sync_copy(acc_sh.at[pl.ds(base, RPT)],
                        seg_out.at[pl.ds(base, RPT)])

    @pl.when(c == 1)
    def _():
        pltpu.sync_copy(acc_sh.at[pl.ds(base, RPT)],
                        cnt_out.at[pl.ds(base, RPT)])


def _make_sc_pass_c():
    return pl.kernel(
        _sc_pass_c_body,
        out_type=(
            jax.ShapeDtypeStruct((NC, BATCH, 128), jnp.float32),
            jax.ShapeDtypeStruct((BATCH, 256), jnp.float32),
            jax.ShapeDtypeStruct((BATCH, 128), jnp.float32),
        ),
        mesh=_mesh(),
        scratch_types=[
            pltpu.VMEM_SHARED((ACC_ROWS, 128), jnp.float32),
            pltpu.VMEM((2, IBLK, 1, CHUNK), jnp.int32),
            pltpu.VMEM((2, IBLK, 1, CHUNK), jnp.int32),
            pltpu.VMEM((CHUNK, 128), jnp.float32),
            pltpu.VMEM((CHUNK, 128), jnp.float32),
            pltpu.VMEM((16, 256), jnp.float32),
            pltpu.VMEM((1, CHUNK), jnp.int32),
            pltpu.VMEM((1, 16), jnp.int32),
            pltpu.SemaphoreType.DMA,
            pltpu.SemaphoreType.DMA,
            pltpu.SemaphoreType.DMA,
            pltpu.SemaphoreType.DMA,
            pltpu.SemaphoreType.DMA,
            pltpu.SemaphoreType.DMA,
        ],
    )


def _sc_pass_c_body(p2a, p2b, src3, dst3, q, cnt128, cell3, cell16, zrow,
                    aggb_out, qb_out, cntb_out,
                    acc_sh, idx_s, idx_d, buf0, buf1, qbuf, cidx, cidx16,
                    sem0, sem1, ss0, ss1, st0, st1):
    c = lax.axis_index("c")
    s = lax.axis_index("s")
    w = c * NS + s

    base = s * RPT
    pltpu.sync_copy(zrow, acc_sh.at[pl.ds(base, RPT)])
    plsc.subcore_barrier()

    @pl.when(c == 0)
    def _():
        _pipelined_gather_scatter(p2a, acc_sh, src3, dst3, idx_s, idx_d,
                                  buf0, buf1, sem0, sem1, ss0, ss1,
                                  st0, st1, s)

    @pl.when(c == 1)
    def _():
        _pipelined_gather_scatter(p2b, acc_sh, src3, dst3, idx_s, idx_d,
                                  buf0, buf1, sem0, sem1, ss0, ss1,
                                  st0, st1, s)

    def qc_body(t, carry):
        pltpu.sync_copy(cell16.at[w * 8 + t], cidx16)
        pltpu.async_copy(q.at[cidx16.at[0]], qbuf, sem0).wait()
        pltpu.sync_copy(qbuf, qb_out.at[pl.ds(w * CHUNK + t * 16, 16)])
        pltpu.async_copy(cnt128.at[cidx16.at[0]], buf0.at[pl.ds(0, 16)],
                         sem0).wait()
        pltpu.sync_copy(buf0.at[pl.ds(0, 16)],
                        cntb_out.at[pl.ds(w * CHUNK + t * 16, 16)])
        return carry
    lax.fori_loop(0, 8, qc_body, 0)

    plsc.subcore_barrier()

    for t in range(2):
        ch = s * 2 + t
        pltpu.sync_copy(cell3.at[ch], cidx)
        pltpu.async_copy(acc_sh.at[cidx.at[0]], buf1, sem1).wait()
        pltpu.sync_copy(buf1, aggb_out.at[c, pl.ds(ch * CHUNK, CHUNK)])


def _tc_b_body(seg_ref, cnt_ref, cx_ref, wl1_ref, wr1_ref, b1_ref,
               wl2a_ref, wl2b_ref, wr2_ref, b2_ref,
               p2a_ref, p2b_ref, q_ref):
    cnt = jnp.maximum(cnt_ref[:, 0:1], 1.0)
    agg = seg_ref[...] / cnt
    h1 = jnp.dot(agg, wl1_ref[...], preferred_element_type=jnp.float32)
    h1 += jnp.dot(cx_ref[...], wr1_ref[...], preferred_element_type=jnp.float32)
    h1 = jnp.maximum(h1 + b1_ref[...], 0.0)
    p2a_ref[...] = jnp.dot(h1, wl2a_ref[...], preferred_element_type=jnp.float32)
    p2b_ref[...] = jnp.dot(h1, wl2b_ref[...], preferred_element_type=jnp.float32)
    q_ref[...] = (jnp.dot(h1, wr2_ref[...], preferred_element_type=jnp.float32)
                  + b2_ref[...])


def _tc_pass_b(seg, cnt128, cx_pad, wl1, wr1, b1, wl2a, wl2b, wr2, b2):
    R = 632
    n = ACC_ROWS // R
    return pl.pallas_call(
        _tc_b_body,
        grid=(n,),
        in_specs=[
            pl.BlockSpec((R, D_IN), lambda i: (i, 0)),
            pl.BlockSpec((R, 128), lambda i: (i, 0)),
            pl.BlockSpec((R, D_IN), lambda i: (i, 0)),
            pl.BlockSpec((D_IN, 1024), lambda i: (0, 0)),
            pl.BlockSpec((D_IN, 1024), lambda i: (0, 0)),
            pl.BlockSpec((1, 1024), lambda i: (0, 0)),
            pl.BlockSpec((1024, 128), lambda i: (0, 0)),
            pl.BlockSpec((1024, 128), lambda i: (0, 0)),
            pl.BlockSpec((1024, 256), lambda i: (0, 0)),
            pl.BlockSpec((1, 256), lambda i: (0, 0)),
        ],
        out_specs=[
            pl.BlockSpec((R, 128), lambda i: (i, 0)),
            pl.BlockSpec((R, 128), lambda i: (i, 0)),
            pl.BlockSpec((R, 256), lambda i: (i, 0)),
        ],
        out_shape=[
            jax.ShapeDtypeStruct((ACC_ROWS, 128), jnp.float32),
            jax.ShapeDtypeStruct((ACC_ROWS, 128), jnp.float32),
            jax.ShapeDtypeStruct((ACC_ROWS, 256), jnp.float32),
        ],
    )(seg, cnt128, cx_pad, wl1, wr1, b1, wl2a, wl2b, wr2, b2)


def _elu(x):
    return jnp.where(x > 0.0, x, jnp.exp(x) - 1.0)


def _tc_d_body(dx_ref, qb_ref, aggb_ref, cntb_ref,
               wde_ref, bde_ref,
               wg1_ref, bg1_ref, wg2_ref, bg2_ref, wg3_ref, bg3_ref,
               out_ref):
    dh = jnp.dot(dx_ref[...], wde_ref[...], preferred_element_type=jnp.float32)
    dh = jnp.maximum(dh + bde_ref[...], 0.0)
    cnt = jnp.maximum(cntb_ref[:, 0:1], 1.0)
    agg2 = jnp.concatenate([aggb_ref[0], aggb_ref[1]], axis=1) / cnt
    h2 = jnp.maximum(agg2 + qb_ref[...], 0.0)
    x = jnp.concatenate([dh, h2], axis=1)
    x = _elu(jnp.dot(x, wg1_ref[...], preferred_element_type=jnp.float32)
             + bg1_ref[...])
    x = _elu(jnp.dot(x, wg2_ref[...], preferred_element_type=jnp.float32)
             + bg2_ref[...])
    out_ref[...] = (jnp.dot(x, wg3_ref[...], preferred_element_type=jnp.float32)
                    + bg3_ref[...])


def _tc_pass_d(dx, qb, aggb, cntb, wde, bde,
               wg1, bg1, wg2, bg2, wg3p, bg3p):
    R = 512
    n = BATCH // R
    return pl.pallas_call(
        _tc_d_body,
        grid=(n,),
        in_specs=[
            pl.BlockSpec((R, D_IN), lambda i: (i, 0)),
            pl.BlockSpec((R, 256), lambda i: (i, 0)),
            pl.BlockSpec((NC, R, 128), lambda i: (0, i, 0)),
            pl.BlockSpec((R, 128), lambda i: (i, 0)),
            pl.BlockSpec((D_IN, 256), lambda i: (0, 0)),
            pl.BlockSpec((1, 256), lambda i: (0, 0)),
            pl.BlockSpec((512, 512), lambda i: (0, 0)),
            pl.BlockSpec((1, 512), lambda i: (0, 0)),
            pl.BlockSpec((512, 512), lambda i: (0, 0)),
            pl.BlockSpec((1, 512), lambda i: (0, 0)),
            pl.BlockSpec((512, 128), lambda i: (0, 0)),
            pl.BlockSpec((1, 128), lambda i: (0, 0)),
        ],
        out_specs=pl.BlockSpec((R, 128), lambda i: (i, 0)),
        out_shape=jax.ShapeDtypeStruct((BATCH, 128), jnp.float32),
    )(dx, qb, aggb, cntb, wde, bde, wg1, bg1, wg2, bg2, wg3p, bg3p)


@jax.jit
def kernel(drug_x, cell_x, W_demb, b_demb, Wl1, Wr1, b1, Wl2, Wr2, b2,
           Wg1, bg1, Wg2, bg2, Wg3, bg3, cell_edges, cell, drug):
    f32 = jnp.float32
    pad_e = E_PAD - E_RAW
    src3 = jnp.concatenate(
        [cell_edges[0], jnp.zeros((pad_e,), jnp.int32)]).reshape(NCHUNK, 1, CHUNK)
    dst3 = jnp.concatenate(
        [cell_edges[1], jnp.full((pad_e,), DUMMY, jnp.int32)]).reshape(NCHUNK, 1, CHUNK)
    drug3 = drug.reshape(NW, 1, CHUNK)
    cell3 = cell.reshape(NW, 1, CHUNK)
    cell16 = cell.reshape(NW * 8, 1, 16)
    cx_pad = jnp.concatenate(
        [cell_x, jnp.zeros((ACC_ROWS - N_NODE, D_IN), f32)], axis=0)
    zrow = jnp.zeros((RPT, D_IN), f32)
    ones128 = jnp.ones((CHUNK, 128), f32)
    wl2a = Wl2[:, :128]
    wl2b = Wl2[:, 128:]
    wg3p = jnp.concatenate([Wg3, jnp.zeros((512, 127), f32)], axis=1)
    bg3p = jnp.concatenate([bg3, jnp.zeros((127,), f32)]).reshape(1, 128)

    seg, cnt128, dx = _make_sc_pass_a()(cx_pad, src3, dst3, drug_x, drug3,
                                        ones128, zrow)

    p2a, p2b, q = _tc_pass_b(
        seg, cnt128, cx_pad, Wl1, Wr1, b1.reshape(1, 1024), wl2a, wl2b,
        Wr2, b2.reshape(1, 256))

    aggb, qb, cntb = _make_sc_pass_c()(
        p2a, p2b, src3, dst3, q, cnt128, cell3, cell16, zrow)

    out = _tc_pass_d(dx, qb, aggb, cntb,
                     W_demb, b_demb.reshape(1, 256),
                     Wg1, bg1.reshape(1, 512), Wg2, bg2.reshape(1, 512),
                     wg3p, bg3p)
    return out[:, :1]


# speedup vs baseline: 1.0492x; 1.0022x over previous
import jax
import jax.numpy as jnp
from jax import lax
from jax.experimental import pallas as pl
from jax.experimental.pallas import tpu as pltpu
from jax.experimental.pallas import tpu_sc as plsc

N_NODE = 10000
D_IN = 128
E_RAW = 320000
BATCH = 4096

NC = 2
NS = 16
NW = NC * NS
CHUNK = 128

K2 = 160
NCHUNK = NS * K2
E_PAD = NCHUNK * CHUNK
IBLK = 16
NBLK = K2 // IBLK
ACC_ROWS = 10112
RPT = ACC_ROWS // NS
DUMMY = N_NODE


def _mesh():
    return plsc.VectorSubcoreMesh(
        core_axis_name="c", subcore_axis_name="s", num_cores=NC, num_subcores=NS)


def _pipelined_gather_scatter(tbl, acc, src3, dst3, idx_s, idx_d,
                              buf0, buf1, sem0, sem1, ss0, ss1,
                              st0, st1, s):
    def issue(j_blk, cur, buf, sa, sb):
        del sb
        pltpu.async_copy(tbl.at[idx_s.at[cur, j_blk, 0]], buf, sa)

    def wait(j_blk, cur, buf, sa, sb):
        del sb
        pltpu.make_async_copy(tbl.at[idx_s.at[cur, j_blk, 0]], buf, sa).wait()

    def scat(j_blk, cur, buf):
        pltpu.sync_copy(buf, acc.at[idx_d.at[cur, j_blk, 0]], add=True)

    pltpu.sync_copy(src3.at[pl.ds(s * K2, IBLK)], idx_s.at[0])
    pltpu.sync_copy(dst3.at[pl.ds(s * K2, IBLK)], idx_d.at[0])
    issue(0, 0, buf0, sem0, ss0)

    for blk in range(NBLK):
        cur = blk % 2
        nxt = 1 - cur
        if blk + 1 < NBLK:
            off = s * K2 + (blk + 1) * IBLK
            pltpu.async_copy(src3.at[pl.ds(off, IBLK)], idx_s.at[nxt], st0)
            pltpu.async_copy(dst3.at[pl.ds(off, IBLK)], idx_d.at[nxt], st1)

        def pair(t, carry):
            issue(2 * t + 1, cur, buf1, sem1, ss1)
            wait(2 * t, cur, buf0, sem0, ss0)
            scat(2 * t, cur, buf0)

            @pl.when(t < IBLK // 2 - 1)
            def _():
                issue(2 * t + 2, cur, buf0, sem0, ss0)
            wait(2 * t + 1, cur, buf1, sem1, ss1)
            scat(2 * t + 1, cur, buf1)
            return carry
        lax.fori_loop(0, IBLK // 2, pair, 0)

        if blk + 1 < NBLK:
            off = s * K2 + (blk + 1) * IBLK
            pltpu.make_async_copy(
                src3.at[pl.ds(off, IBLK)], idx_s.at[nxt], st0).wait()
            pltpu.make_async_copy(
                dst3.at[pl.ds(off, IBLK)], idx_d.at[nxt], st1).wait()
            issue(0, nxt, buf0, sem0, ss0)


def _make_sc_pass_a():
    return pl.kernel(
        _sc_pass_a_body,
        out_type=(
            jax.ShapeDtypeStruct((ACC_ROWS, D_IN), jnp.float32),
            jax.ShapeDtypeStruct((ACC_ROWS, 128), jnp.float32),
            jax.ShapeDtypeStruct((BATCH, D_IN), jnp.float32),
        ),
        mesh=_mesh(),
        scratch_types=[
            pltpu.VMEM_SHARED((ACC_ROWS, D_IN), jnp.float32),
            pltpu.VMEM((2, IBLK, 1, CHUNK), jnp.int32),
            pltpu.VMEM((2, IBLK, 1, CHUNK), jnp.int32),
            pltpu.VMEM((CHUNK, D_IN), jnp.float32),
            pltpu.VMEM((CHUNK, D_IN), jnp.float32),
            pltpu.VMEM((1, CHUNK), jnp.int32),
            pltpu.SemaphoreType.DMA,
            pltpu.SemaphoreType.DMA,
            pltpu.SemaphoreType.DMA,
            pltpu.SemaphoreType.DMA,
            pltpu.SemaphoreType.DMA,
            pltpu.SemaphoreType.DMA,
        ],
    )


def _sc_pass_a_body(cellx, src3, dst3, drugx, drug3, ones128, zrow,
                    seg_out, cnt_out, dx_out,
                    acc_sh, idx_s, idx_d, buf0, buf1, didx,
                    sem0, sem1, ss0, ss1, st0, st1):
    c = lax.axis_index("c")
    s = lax.axis_index("s")

    base = s * RPT
    pltpu.sync_copy(zrow, acc_sh.at[pl.ds(base, RPT)])
    plsc.subcore_barrier()

    @pl.when(c == 0)
    def _():
        _pipelined_gather_scatter(cellx, acc_sh, src3, dst3, idx_s, idx_d,
                                  buf0, buf1, sem0, sem1, ss0, ss1,
                                  st0, st1, s)

    @pl.when(c == 1)
    def _():
        pltpu.sync_copy(ones128, buf0)
        for blk in range(NBLK):
            pltpu.sync_copy(dst3.at[pl.ds(s * K2 + blk * IBLK, IBLK)],
                            idx_d.at[0])
            for u in range(IBLK):
                pltpu.async_copy(buf0, acc_sh.at[idx_d.at[0, u, 0]], sem0,
                                 add=True)
            for u in range(IBLK):
                pltpu.make_async_copy(
                    buf0, acc_sh.at[idx_d.at[0, u, 0]], sem0).wait()
        for t in range(2):
            ch = s * 2 + t
            pltpu.sync_copy(drug3.at[ch], didx)
            pltpu.async_copy(drugx.at[didx.at[0]], buf1, sem1).wait()
            pltpu.sync_copy(buf1, dx_out.at[pl.ds(ch * CHUNK, CHUNK)])

    plsc.subcore_barrier()

    @pl.when(c == 0)
    def _():
        pltpu.sync_copy(acc_sh.at[pl.ds(base, RPT)],
                        seg_out.at[pl.ds(base, RPT)])

    @pl.when(c == 1)
    def _():
        pltpu.sync_copy(acc_sh.at[pl.ds(base, RPT)],
                        cnt_out.at[pl.ds(base, RPT)])


def _make_sc_pass_c():
    return pl.kernel(
        _sc_pass_c_body,
        out_type=(
            jax.ShapeDtypeStruct((NC, BATCH, 128), jnp.float32),
            jax.ShapeDtypeStruct((BATCH, 256), jnp.float32),
            jax.ShapeDtypeStruct((BATCH, 128), jnp.float32),
        ),
        mesh=_mesh(),
        scratch_types=[
            pltpu.VMEM_SHARED((ACC_ROWS, 128), jnp.float32),
            pltpu.VMEM((2, IBLK, 1, CHUNK), jnp.int32),
            pltpu.VMEM((2, IBLK, 1, CHUNK), jnp.int32),
            pltpu.VMEM((CHUNK, 128), jnp.float32),
            pltpu.VMEM((CHUNK, 128), jnp.float32),
            pltpu.VMEM((16, 256), jnp.float32),
            pltpu.VMEM((1, CHUNK), jnp.int32),
            pltpu.VMEM((1, 16), jnp.int32),
            pltpu.SemaphoreType.DMA,
            pltpu.SemaphoreType.DMA,
            pltpu.SemaphoreType.DMA,
            pltpu.SemaphoreType.DMA,
            pltpu.SemaphoreType.DMA,
            pltpu.SemaphoreType.DMA,
        ],
    )


def _sc_pass_c_body(p2a, p2b, src3, dst3, q, cnt128, cell3, cell16, zrow,
                    aggb_out, qb_out, cntb_out,
                    acc_sh, idx_s, idx_d, buf0, buf1, qbuf, cidx, cidx16,
                    sem0, sem1, ss0, ss1, st0, st1):
    c = lax.axis_index("c")
    s = lax.axis_index("s")
    w = c * NS + s

    base = s * RPT
    pltpu.sync_copy(zrow, acc_sh.at[pl.ds(base, RPT)])
    plsc.subcore_barrier()

    @pl.when(c == 0)
    def _():
        _pipelined_gather_scatter(p2a, acc_sh, src3, dst3, idx_s, idx_d,
                                  buf0, buf1, sem0, sem1, ss0, ss1,
                                  st0, st1, s)

    @pl.when(c == 1)
    def _():
        _pipelined_gather_scatter(p2b, acc_sh, src3, dst3, idx_s, idx_d,
                                  buf0, buf1, sem0, sem1, ss0, ss1,
                                  st0, st1, s)

    def qc_body(t, carry):
        pltpu.sync_copy(cell16.at[w * 8 + t], cidx16)
        pltpu.async_copy(q.at[cidx16.at[0]], qbuf, sem0).wait()
        pltpu.sync_copy(qbuf, qb_out.at[pl.ds(w * CHUNK + t * 16, 16)])
        pltpu.async_copy(cnt128.at[cidx16.at[0]], buf0.at[pl.ds(0, 16)],
                         sem0).wait()
        pltpu.sync_copy(buf0.at[pl.ds(0, 16)],
                        cntb_out.at[pl.ds(w * CHUNK + t * 16, 16)])
        return carry
    lax.fori_loop(0, 8, qc_body, 0)

    plsc.subcore_barrier()

    for t in range(2):
        ch = s * 2 + t
        pltpu.sync_copy(cell3.at[ch], cidx)
        pltpu.async_copy(acc_sh.at[cidx.at[0]], buf1, sem1).wait()
        pltpu.sync_copy(buf1, aggb_out.at[c, pl.ds(ch * CHUNK, CHUNK)])


def _tc_b_body(seg_ref, cnt_ref, cx_ref, wl1_ref, wr1_ref, b1_ref,
               wl2a_ref, wl2b_ref, wr2_ref, b2_ref,
               p2a_ref, p2b_ref, q_ref):
    cnt = jnp.maximum(cnt_ref[:, 0:1], 1.0)
    agg = seg_ref[...] / cnt
    h1 = jnp.dot(agg, wl1_ref[...], preferred_element_type=jnp.float32)
    h1 += jnp.dot(cx_ref[...], wr1_ref[...], preferred_element_type=jnp.float32)
    h1 = jnp.maximum(h1 + b1_ref[...], 0.0)
    p2a_ref[...] = jnp.dot(h1, wl2a_ref[...], preferred_element_type=jnp.float32)
    p2b_ref[...] = jnp.dot(h1, wl2b_ref[...], preferred_element_type=jnp.float32)
    q_ref[...] = (jnp.dot(h1, wr2_ref[...], preferred_element_type=jnp.float32)
                  + b2_ref[...])


def _tc_pass_b(seg, cnt128, cx_pad, wl1, wr1, b1, wl2a, wl2b, wr2, b2):
    R = 632
    n = ACC_ROWS // R
    return pl.pallas_call(
        _tc_b_body,
        grid=(n,),
        in_specs=[
            pl.BlockSpec((R, D_IN), lambda i: (i, 0)),
            pl.BlockSpec((R, 128), lambda i: (i, 0)),
            pl.BlockSpec((R, D_IN), lambda i: (i, 0)),
            pl.BlockSpec((D_IN, 1024), lambda i: (0, 0)),
            pl.BlockSpec((D_IN, 1024), lambda i: (0, 0)),
            pl.BlockSpec((1, 1024), lambda i: (0, 0)),
            pl.BlockSpec((1024, 128), lambda i: (0, 0)),
            pl.BlockSpec((1024, 128), lambda i: (0, 0)),
            pl.BlockSpec((1024, 256), lambda i: (0, 0)),
            pl.BlockSpec((1, 256), lambda i: (0, 0)),
        ],
        out_specs=[
            pl.BlockSpec((R, 128), lambda i: (i, 0)),
            pl.BlockSpec((R, 128), lambda i: (i, 0)),
            pl.BlockSpec((R, 256), lambda i: (i, 0)),
        ],
        out_shape=[
            jax.ShapeDtypeStruct((ACC_ROWS, 128), jnp.float32),
            jax.ShapeDtypeStruct((ACC_ROWS, 128), jnp.float32),
            jax.ShapeDtypeStruct((ACC_ROWS, 256), jnp.float32),
        ],
    )(seg, cnt128, cx_pad, wl1, wr1, b1, wl2a, wl2b, wr2, b2)


def _elu(x):
    return jnp.where(x > 0.0, x, jnp.exp(x) - 1.0)


def _tc_d_body(dx_ref, qb_ref, aggb_ref, cntb_ref,
               wde_ref, bde_ref,
               wg1_ref, bg1_ref, wg2_ref, bg2_ref, wg3_ref, bg3_ref,
               out_ref):
    dh = jnp.dot(dx_ref[...], wde_ref[...], preferred_element_type=jnp.float32)
    dh = jnp.maximum(dh + bde_ref[...], 0.0)
    cnt = jnp.maximum(cntb_ref[:, 0:1], 1.0)
    agg2 = jnp.concatenate([aggb_ref[0], aggb_ref[1]], axis=1) / cnt
    h2 = jnp.maximum(agg2 + qb_ref[...], 0.0)
    x = jnp.concatenate([dh, h2], axis=1)
    x = _elu(jnp.dot(x, wg1_ref[...], preferred_element_type=jnp.float32)
             + bg1_ref[...])
    x = _elu(jnp.dot(x, wg2_ref[...], preferred_element_type=jnp.float32)
             + bg2_ref[...])
    out_ref[...] = (jnp.dot(x, wg3_ref[...], preferred_element_type=jnp.float32)
                    + bg3_ref[...])


def _tc_pass_d(dx, qb, aggb, cntb, wde, bde,
               wg1, bg1, wg2, bg2, wg3p, bg3p):
    R = 512
    n = BATCH // R
    return pl.pallas_call(
        _tc_d_body,
        grid=(n,),
        in_specs=[
            pl.BlockSpec((R, D_IN), lambda i: (i, 0)),
            pl.BlockSpec((R, 256), lambda i: (i, 0)),
            pl.BlockSpec((NC, R, 128), lambda i: (0, i, 0)),
            pl.BlockSpec((R, 128), lambda i: (i, 0)),
            pl.BlockSpec((D_IN, 256), lambda i: (0, 0)),
            pl.BlockSpec((1, 256), lambda i: (0, 0)),
            pl.BlockSpec((512, 512), lambda i: (0, 0)),
            pl.BlockSpec((1, 512), lambda i: (0, 0)),
            pl.BlockSpec((512, 512), lambda i: (0, 0)),
            pl.BlockSpec((1, 512), lambda i: (0, 0)),
            pl.BlockSpec((512, 128), lambda i: (0, 0)),
            pl.BlockSpec((1, 128), lambda i: (0, 0)),
        ],
        out_specs=pl.BlockSpec((R, 128), lambda i: (i, 0)),
        out_shape=jax.ShapeDtypeStruct((BATCH, 128), jnp.float32),
    )(dx, qb, aggb, cntb, wde, bde, wg1, bg1, wg2, bg2, wg3p, bg3p)


@jax.jit
def kernel(drug_x, cell_x, W_demb, b_demb, Wl1, Wr1, b1, Wl2, Wr2, b2,
           Wg1, bg1, Wg2, bg2, Wg3, bg3, cell_edges, cell, drug):
    f32 = jnp.float32
    pad_e = E_PAD - E_RAW
    src3 = jnp.concatenate(
        [cell_edges[0], jnp.zeros((pad_e,), jnp.int32)]).reshape(NCHUNK, 1, CHUNK)
    dst3 = jnp.concatenate(
        [cell_edges[1], jnp.full((pad_e,), DUMMY, jnp.int32)]).reshape(NCHUNK, 1, CHUNK)
    drug3 = drug.reshape(NW, 1, CHUNK)
    cell3 = cell.reshape(NW, 1, CHUNK)
    cell16 = cell.reshape(NW * 8, 1, 16)
    cx_pad = jnp.concatenate(
        [cell_x, jnp.zeros((ACC_ROWS - N_NODE, D_IN), f32)], axis=0)
    zrow = jnp.zeros((RPT, D_IN), f32)
    ones128 = jnp.ones((CHUNK, 128), f32)
    wl2a = Wl2[:, :128]
    wl2b = Wl2[:, 128:]
    wg3p = jnp.concatenate([Wg3, jnp.zeros((512, 127), f32)], axis=1)
    bg3p = jnp.concatenate([bg3, jnp.zeros((127,), f32)]).reshape(1, 128)

    seg, cnt128, dx = _make_sc_pass_a()(cx_pad, src3, dst3, drug_x, drug3,
                                        ones128, zrow)

    p2a, p2b, q = _tc_pass_b(
        seg, cnt128, cx_pad, Wl1, Wr1, b1.reshape(1, 1024), wl2a, wl2b,
        Wr2, b2.reshape(1, 256))

    aggb, qb, cntb = _make_sc_pass_c()(
        p2a, p2b, src3, dst3, q, cnt128, cell3, cell16, zrow)

    out = _tc_pass_d(dx, qb, aggb, cntb,
                     W_demb, b_demb.reshape(1, 256),
                     Wg1, bg1.reshape(1, 512), Wg2, bg2.reshape(1, 512),
                     wg3p, bg3p)
    return out[:, :1]
